# baseline (speedup 1.0000x reference)
import functools

import jax
import jax.numpy as jnp
from jax import lax
from jax.experimental import pallas as pl
from jax.experimental.pallas import tpu as pltpu
from jax.experimental.pallas import tpu_sc as plsc

_INFO = plsc.get_sparse_core_info()
_NC = _INFO.num_cores
_NS = _INFO.num_subcores
_NW = _NC * _NS
_L = 16
_R = 8


def _make_lookup(dim, batch):
    assert batch % (_NW * 2 * _R) == 0
    b_per_w = batch // _NW
    n_pairs = b_per_w // (2 * _R)
    mesh = plsc.VectorSubcoreMesh(core_axis_name="c", subcore_axis_name="s")

    @functools.partial(
        pl.kernel,
        mesh=mesh,
        out_type=jax.ShapeDtypeStruct((dim, batch), jnp.float32),
        scratch_types=[
            pltpu.VMEM((b_per_w + _L,), jnp.int32),
            pltpu.VMEM((_R, dim, 128), jnp.float32),
            pltpu.VMEM((_R, dim, 128), jnp.float32),
            pltpu.VMEM((dim, b_per_w), jnp.float32),
            pltpu.SemaphoreType.DMA,
            pltpu.SemaphoreType.DMA,
        ],
        compiler_params=pltpu.CompilerParams(needs_layout_passes=False),
    )
    def lookup(idx_hbm, tab_hbm, out_hbm, idx_v, ring_a, ring_b, outt_v,
               sem_a, sem_b):
        wid = lax.axis_index("s") * _NC + lax.axis_index("c")
        base = wid * b_per_w
        pltpu.sync_copy(
            idx_hbm.at[pl.ds(base, b_per_w)], idx_v.at[pl.ds(0, b_per_w)]
        )

        lanes = lax.iota(jnp.int32, _L)

        def fire(k0, ring, sem):
            v16 = idx_v[pl.ds(k0, _L)]
            for j in range(_R):
                col0 = pl.multiple_of(
                    lax.shift_left(
                        lax.shift_right_logical(v16[j], 7), 7
                    ),
                    128,
                )
                pltpu.async_copy(
                    tab_hbm.at[:, pl.ds(col0, 128)], ring.at[j], sem
                )

        def drain(ring, sem):
            for j in range(_R):
                pltpu.make_async_copy(
                    tab_hbm.at[:, pl.ds(0, 128)], ring.at[j], sem
                ).wait()

        def extract(k0, ring):
            for j in range(_R):
                lsplat = plsc.load_gather(
                    idx_v, [jnp.full((_L,), k0 + j, jnp.int32)]
                )
                lsplat = lax.bitwise_and(lsplat, 127)
                ksplat = jnp.full((_L,), k0 + j, jnp.int32)
                jsplat = jnp.full((_L,), j, jnp.int32)
                for h in range(dim // _L):
                    vals = plsc.load_gather(
                        ring, [jsplat, lanes + h * _L, lsplat]
                    )
                    plsc.store_scatter(
                        outt_v, [lanes + h * _L, ksplat], vals
                    )

        def body(p, carry):
            k0 = p * 2 * _R
            fire(k0, ring_a, sem_a)

            @pl.when(p > 0)
            def _prev():
                drain(ring_b, sem_b)
                extract(k0 - _R, ring_b)

            fire(k0 + _R, ring_b, sem_b)
            drain(ring_a, sem_a)
            extract(k0, ring_a)
            return carry

        lax.fori_loop(0, n_pairs, body, 0)
        drain(ring_b, sem_b)
        extract(b_per_w - _R, ring_b)

        pltpu.sync_copy(outt_v, out_hbm.at[:, pl.ds(base, b_per_w)])

    return lookup


@jax.jit
def kernel(skill_id, emb_weight):
    batch = skill_id.shape[0]
    n_rows, dim = emb_weight.shape
    out_t = _make_lookup(dim, batch)(
        skill_id.astype(jnp.int32), emb_weight.T
    )
    return out_t.T
